# baseline (device time: 64556 ns/iter reference)
import jax
import jax.numpy as jnp
from jax import lax
from jax.experimental import pallas as pl
from jax.experimental.pallas import tpu as pltpu

S = 8

_DNUMS = (((0,), (0,)), ((), ()))
_MESH = pl.DeviceIdType.MESH


def kernel(x, dy):
    m, d = x.shape
    _, f = dy.shape
    half = d // 2
    fq = f // 4
    c = fq // S

    def body(x_ref, dy_ref, out_ref, part_ref, rxx_ref,
             sx_send, sx_recv, sy1_send, sy1_recv, sz1_send, sz1_recv,
             s2_send, s2_recv):
        my_x = lax.axis_index("x")
        my_y = lax.axis_index("y")
        my_z = lax.axis_index("z")
        xn = (1 - my_x, my_y, my_z)
        yn = (my_x, 1 - my_y, my_z)
        zn = (my_x, my_y, 1 - my_z)
        q = 2 * my_y + my_z
        qy = 2 * (1 - my_y) + my_z
        qz = 2 * my_y + (1 - my_z)
        qd = 2 * (1 - my_y) + (1 - my_z)
        other_start = (1 - my_x) * half

        barrier_sem = pltpu.get_barrier_semaphore()
        for nbr in (xn, yn, zn):
            pl.semaphore_signal(barrier_sem, inc=1, device_id=nbr,
                                device_id_type=_MESH)
        pl.semaphore_wait(barrier_sem, 3)

        def x_rdma(s):
            return pltpu.make_async_remote_copy(
                src_ref=part_ref.at[pl.ds(other_start, half), pl.ds(s * c, c)],
                dst_ref=rxx_ref.at[:, pl.ds(s * c, c)],
                send_sem=sx_send.at[s], recv_sem=sx_recv.at[s],
                device_id=xn, device_id_type=_MESH)

        def r1_rdma(s, nbr, send_sems, recv_sems):
            return pltpu.make_async_remote_copy(
                src_ref=out_ref.at[:, pl.ds(q * fq + s * c, c)],
                dst_ref=out_ref.at[:, pl.ds(q * fq + s * c, c)],
                send_sem=send_sems.at[s], recv_sem=recv_sems.at[s],
                device_id=nbr, device_id_type=_MESH)

        def r1_recv(s, slot, recv_sems):
            return pltpu.make_async_remote_copy(
                src_ref=out_ref.at[:, pl.ds(slot * fq + s * c, c)],
                dst_ref=out_ref.at[:, pl.ds(slot * fq + s * c, c)],
                send_sem=s2_send.at[s], recv_sem=recv_sems.at[s],
                device_id=xn, device_id_type=_MESH)

        def relay_rdma(s):
            slot = qz if s % 2 == 0 else qy
            nbr = yn if s % 2 == 0 else zn
            return pltpu.make_async_remote_copy(
                src_ref=out_ref.at[:, pl.ds(slot * fq + s * c, c)],
                dst_ref=out_ref.at[:, pl.ds(slot * fq + s * c, c)],
                send_sem=s2_send.at[s], recv_sem=s2_recv.at[s],
                device_id=nbr, device_id_type=_MESH)

        for s in range(S):
            for i in range(4):
                @pl.when(q == i)
                def _(s=s, i=i):
                    part_ref[:, s * c:(s + 1) * c] = lax.dot_general(
                        x_ref[...],
                        dy_ref[:, i * fq + s * c: i * fq + (s + 1) * c],
                        dimension_numbers=_DNUMS,
                        preferred_element_type=jnp.float32)
            x_rdma(s).start()

        for s in range(S):
            x_rdma(s).wait_recv()
            for i in range(4):
                for j in range(2):
                    @pl.when((q == i) & (my_x == j))
                    def _(s=s, i=i, j=j):
                        own = j * half
                        out_ref[:, i * fq + s * c: i * fq + (s + 1) * c] = (
                            part_ref[own:own + half, s * c:(s + 1) * c]
                            + rxx_ref[:, s * c:(s + 1) * c])
            r1_rdma(s, yn, sy1_send, sy1_recv).start()
            r1_rdma(s, zn, sz1_send, sz1_recv).start()

        for s in range(S):
            if s % 2 == 0:
                r1_recv(s, qz, sz1_recv).wait_recv()
            else:
                r1_recv(s, qy, sy1_recv).wait_recv()
            relay_rdma(s).start()

        for s in range(S):
            if s % 2 == 0:
                r1_recv(s, qy, sy1_recv).wait_recv()
            else:
                r1_recv(s, qz, sz1_recv).wait_recv()
            pltpu.make_async_remote_copy(
                src_ref=out_ref.at[:, pl.ds(qd * fq + s * c, c)],
                dst_ref=out_ref.at[:, pl.ds(qd * fq + s * c, c)],
                send_sem=s2_send.at[s], recv_sem=s2_recv.at[s],
                device_id=yn if s % 2 == 0 else zn,
                device_id_type=_MESH).wait_recv()
            x_rdma(s).wait_send()
            r1_rdma(s, yn, sy1_send, sy1_recv).wait_send()
            r1_rdma(s, zn, sz1_send, sz1_recv).wait_send()
            relay_rdma(s).wait_send()

    return pl.pallas_call(
        body,
        out_shape=jax.ShapeDtypeStruct((half, f), jnp.float32),
        in_specs=[
            pl.BlockSpec(memory_space=pltpu.VMEM),
            pl.BlockSpec(memory_space=pltpu.VMEM),
        ],
        out_specs=pl.BlockSpec(memory_space=pltpu.VMEM),
        scratch_shapes=[
            pltpu.VMEM((d, fq), jnp.float32),
            pltpu.VMEM((half, fq), jnp.float32),
            pltpu.SemaphoreType.DMA((S,)),
            pltpu.SemaphoreType.DMA((S,)),
            pltpu.SemaphoreType.DMA((S,)),
            pltpu.SemaphoreType.DMA((S,)),
            pltpu.SemaphoreType.DMA((S,)),
            pltpu.SemaphoreType.DMA((S,)),
            pltpu.SemaphoreType.DMA((S,)),
            pltpu.SemaphoreType.DMA((S,)),
        ],
        compiler_params=pltpu.CompilerParams(
            collective_id=0, vmem_limit_bytes=100 * 1024 * 1024
        ),
    )(x, dy)


# device time: 29356 ns/iter; 2.1991x vs baseline; 2.1991x over previous
import os

import jax
import jax.numpy as jnp
from jax import lax
from jax.experimental import pallas as pl
from jax.experimental.pallas import tpu as pltpu

S = 8
_ABLATE = os.environ.get("KERNEL_ABLATE", "")

_DNUMS = (((0,), (0,)), ((), ()))
_MESH = pl.DeviceIdType.MESH


def kernel(x, dy):
    m, d = x.shape
    _, f = dy.shape
    half = d // 2
    fq = f // 4
    c = fq // S

    def body(x_ref, dy_ref, out_ref, part_ref, rxx_ref,
             sx_send, sx_recv, sy1_send, sy1_recv, sz1_send, sz1_recv,
             s2_send, s2_recv):
        my_x = lax.axis_index("x")
        my_y = lax.axis_index("y")
        my_z = lax.axis_index("z")
        xn = (1 - my_x, my_y, my_z)
        yn = (my_x, 1 - my_y, my_z)
        zn = (my_x, my_y, 1 - my_z)
        q = 2 * my_y + my_z
        qy = 2 * (1 - my_y) + my_z
        qz = 2 * my_y + (1 - my_z)
        qd = 2 * (1 - my_y) + (1 - my_z)
        other_start = (1 - my_x) * half

        barrier_sem = pltpu.get_barrier_semaphore()
        for nbr in (xn, yn, zn):
            pl.semaphore_signal(barrier_sem, inc=1, device_id=nbr,
                                device_id_type=_MESH)
        pl.semaphore_wait(barrier_sem, 3)

        def x_rdma(s):
            return pltpu.make_async_remote_copy(
                src_ref=part_ref.at[pl.ds(other_start, half), pl.ds(s * c, c)],
                dst_ref=rxx_ref.at[:, pl.ds(s * c, c)],
                send_sem=sx_send.at[s], recv_sem=sx_recv.at[s],
                device_id=xn, device_id_type=_MESH)

        def r1_rdma(s, nbr, send_sems, recv_sems):
            return pltpu.make_async_remote_copy(
                src_ref=out_ref.at[:, pl.ds(q * fq + s * c, c)],
                dst_ref=out_ref.at[:, pl.ds(q * fq + s * c, c)],
                send_sem=send_sems.at[s], recv_sem=recv_sems.at[s],
                device_id=nbr, device_id_type=_MESH)

        def r1_recv(s, slot, recv_sems):
            return pltpu.make_async_remote_copy(
                src_ref=out_ref.at[:, pl.ds(slot * fq + s * c, c)],
                dst_ref=out_ref.at[:, pl.ds(slot * fq + s * c, c)],
                send_sem=s2_send.at[s], recv_sem=recv_sems.at[s],
                device_id=xn, device_id_type=_MESH)

        def relay_rdma(s):
            slot = qz if s % 2 == 0 else qy
            nbr = yn if s % 2 == 0 else zn
            return pltpu.make_async_remote_copy(
                src_ref=out_ref.at[:, pl.ds(slot * fq + s * c, c)],
                dst_ref=out_ref.at[:, pl.ds(slot * fq + s * c, c)],
                send_sem=s2_send.at[s], recv_sem=s2_recv.at[s],
                device_id=nbr, device_id_type=_MESH)

        for s in range(S):
            for i in range(4):
                @pl.when(q == i)
                def _(s=s, i=i):
                    part_ref[:, s * c:(s + 1) * c] = lax.dot_general(
                        x_ref[...],
                        dy_ref[:, i * fq + s * c: i * fq + (s + 1) * c],
                        dimension_numbers=_DNUMS,
                        preferred_element_type=jnp.float32)
            if not _ABLATE:
                x_rdma(s).start()

        if _ABLATE == "compute":
            for s in range(S):
                for i in range(4):
                    for j in range(2):
                        @pl.when((q == i) & (my_x == j))
                        def _(s=s, i=i, j=j):
                            own = j * half
                            out_ref[:, i * fq + s * c: i * fq + (s + 1) * c] = (
                                part_ref[own:own + half, s * c:(s + 1) * c]
                                + rxx_ref[:, s * c:(s + 1) * c])
            return

        for s in range(S):
            x_rdma(s).wait_recv()
            for i in range(4):
                for j in range(2):
                    @pl.when((q == i) & (my_x == j))
                    def _(s=s, i=i, j=j):
                        own = j * half
                        out_ref[:, i * fq + s * c: i * fq + (s + 1) * c] = (
                            part_ref[own:own + half, s * c:(s + 1) * c]
                            + rxx_ref[:, s * c:(s + 1) * c])
            r1_rdma(s, yn, sy1_send, sy1_recv).start()
            r1_rdma(s, zn, sz1_send, sz1_recv).start()

        for s in range(S):
            if s % 2 == 0:
                r1_recv(s, qz, sz1_recv).wait_recv()
            else:
                r1_recv(s, qy, sy1_recv).wait_recv()
            relay_rdma(s).start()

        for s in range(S):
            if s % 2 == 0:
                r1_recv(s, qy, sy1_recv).wait_recv()
            else:
                r1_recv(s, qz, sz1_recv).wait_recv()
            pltpu.make_async_remote_copy(
                src_ref=out_ref.at[:, pl.ds(qd * fq + s * c, c)],
                dst_ref=out_ref.at[:, pl.ds(qd * fq + s * c, c)],
                send_sem=s2_send.at[s], recv_sem=s2_recv.at[s],
                device_id=yn if s % 2 == 0 else zn,
                device_id_type=_MESH).wait_recv()
            x_rdma(s).wait_send()
            r1_rdma(s, yn, sy1_send, sy1_recv).wait_send()
            r1_rdma(s, zn, sz1_send, sz1_recv).wait_send()
            relay_rdma(s).wait_send()

    return pl.pallas_call(
        body,
        out_shape=jax.ShapeDtypeStruct((half, f), jnp.float32),
        in_specs=[
            pl.BlockSpec(memory_space=pltpu.VMEM),
            pl.BlockSpec(memory_space=pltpu.VMEM),
        ],
        out_specs=pl.BlockSpec(memory_space=pltpu.VMEM),
        scratch_shapes=[
            pltpu.VMEM((d, fq), jnp.float32),
            pltpu.VMEM((half, fq), jnp.float32),
            pltpu.SemaphoreType.DMA((S,)),
            pltpu.SemaphoreType.DMA((S,)),
            pltpu.SemaphoreType.DMA((S,)),
            pltpu.SemaphoreType.DMA((S,)),
            pltpu.SemaphoreType.DMA((S,)),
            pltpu.SemaphoreType.DMA((S,)),
            pltpu.SemaphoreType.DMA((S,)),
            pltpu.SemaphoreType.DMA((S,)),
        ],
        compiler_params=pltpu.CompilerParams(
            collective_id=0, vmem_limit_bytes=100 * 1024 * 1024
        ),
    )(x, dy)


# device time: 28618 ns/iter; 2.2558x vs baseline; 1.0258x over previous
import os

import jax
import jax.numpy as jnp
from jax import lax
from jax.experimental import pallas as pl
from jax.experimental.pallas import tpu as pltpu

S = 8
_ABLATE = os.environ.get("KERNEL_ABLATE", "")

_DNUMS = (((0,), (0,)), ((), ()))
_MESH = pl.DeviceIdType.MESH


def kernel(x, dy):
    m, d = x.shape
    _, f = dy.shape
    half = d // 2
    fq = f // 4
    c = fq // S

    def body(x_ref, dy_ref, out_ref, xt_ref, part_ref, rxx_ref,
             sx_send, sx_recv, sy1_send, sy1_recv, sz1_send, sz1_recv,
             s2_send, s2_recv):
        my_x = lax.axis_index("x")
        my_y = lax.axis_index("y")
        my_z = lax.axis_index("z")
        xn = (1 - my_x, my_y, my_z)
        yn = (my_x, 1 - my_y, my_z)
        zn = (my_x, my_y, 1 - my_z)
        q = 2 * my_y + my_z
        qy = 2 * (1 - my_y) + my_z
        qz = 2 * my_y + (1 - my_z)
        qd = 2 * (1 - my_y) + (1 - my_z)
        other_start = (1 - my_x) * half

        barrier_sem = pltpu.get_barrier_semaphore()
        for nbr in (xn, yn, zn):
            pl.semaphore_signal(barrier_sem, inc=1, device_id=nbr,
                                device_id_type=_MESH)
        pl.semaphore_wait(barrier_sem, 3)

        def x_rdma(s):
            return pltpu.make_async_remote_copy(
                src_ref=part_ref.at[pl.ds(other_start, half), pl.ds(s * c, c)],
                dst_ref=rxx_ref.at[:, pl.ds(s * c, c)],
                send_sem=sx_send.at[s], recv_sem=sx_recv.at[s],
                device_id=xn, device_id_type=_MESH)

        def r1_rdma(s, nbr, send_sems, recv_sems):
            return pltpu.make_async_remote_copy(
                src_ref=out_ref.at[:, pl.ds(q * fq + s * c, c)],
                dst_ref=out_ref.at[:, pl.ds(q * fq + s * c, c)],
                send_sem=send_sems.at[s], recv_sem=recv_sems.at[s],
                device_id=nbr, device_id_type=_MESH)

        def r1_recv(s, slot, recv_sems):
            return pltpu.make_async_remote_copy(
                src_ref=out_ref.at[:, pl.ds(slot * fq + s * c, c)],
                dst_ref=out_ref.at[:, pl.ds(slot * fq + s * c, c)],
                send_sem=s2_send.at[s], recv_sem=recv_sems.at[s],
                device_id=xn, device_id_type=_MESH)

        def relay_rdma(s):
            slot = qz if s % 2 == 0 else qy
            nbr = yn if s % 2 == 0 else zn
            return pltpu.make_async_remote_copy(
                src_ref=out_ref.at[:, pl.ds(slot * fq + s * c, c)],
                dst_ref=out_ref.at[:, pl.ds(slot * fq + s * c, c)],
                send_sem=s2_send.at[s], recv_sem=s2_recv.at[s],
                device_id=nbr, device_id_type=_MESH)

        xt_ref[...] = x_ref[...].T

        for s in range(S):
            for i in range(4):
                @pl.when(q == i)
                def _(s=s, i=i):
                    part_ref[:, s * c:(s + 1) * c] = jnp.dot(
                        xt_ref[...],
                        dy_ref[:, i * fq + s * c: i * fq + (s + 1) * c],
                        preferred_element_type=jnp.float32)
            if not _ABLATE:
                x_rdma(s).start()

        if _ABLATE == "compute":
            for s in range(S):
                for i in range(4):
                    for j in range(2):
                        @pl.when((q == i) & (my_x == j))
                        def _(s=s, i=i, j=j):
                            own = j * half
                            out_ref[:, i * fq + s * c: i * fq + (s + 1) * c] = (
                                part_ref[own:own + half, s * c:(s + 1) * c]
                                + rxx_ref[:, s * c:(s + 1) * c])
            return

        for s in range(S):
            x_rdma(s).wait_recv()
            for i in range(4):
                for j in range(2):
                    @pl.when((q == i) & (my_x == j))
                    def _(s=s, i=i, j=j):
                        own = j * half
                        out_ref[:, i * fq + s * c: i * fq + (s + 1) * c] = (
                            part_ref[own:own + half, s * c:(s + 1) * c]
                            + rxx_ref[:, s * c:(s + 1) * c])
            r1_rdma(s, yn, sy1_send, sy1_recv).start()
            r1_rdma(s, zn, sz1_send, sz1_recv).start()

        for s in range(S):
            if s % 2 == 0:
                r1_recv(s, qz, sz1_recv).wait_recv()
            else:
                r1_recv(s, qy, sy1_recv).wait_recv()
            relay_rdma(s).start()

        for s in range(S):
            if s % 2 == 0:
                r1_recv(s, qy, sy1_recv).wait_recv()
            else:
                r1_recv(s, qz, sz1_recv).wait_recv()
            pltpu.make_async_remote_copy(
                src_ref=out_ref.at[:, pl.ds(qd * fq + s * c, c)],
                dst_ref=out_ref.at[:, pl.ds(qd * fq + s * c, c)],
                send_sem=s2_send.at[s], recv_sem=s2_recv.at[s],
                device_id=yn if s % 2 == 0 else zn,
                device_id_type=_MESH).wait_recv()
            x_rdma(s).wait_send()
            r1_rdma(s, yn, sy1_send, sy1_recv).wait_send()
            r1_rdma(s, zn, sz1_send, sz1_recv).wait_send()
            relay_rdma(s).wait_send()

    return pl.pallas_call(
        body,
        out_shape=jax.ShapeDtypeStruct((half, f), jnp.float32),
        in_specs=[
            pl.BlockSpec(memory_space=pltpu.VMEM),
            pl.BlockSpec(memory_space=pltpu.VMEM),
        ],
        out_specs=pl.BlockSpec(memory_space=pltpu.VMEM),
        scratch_shapes=[
            pltpu.VMEM((d, m), jnp.float32),
            pltpu.VMEM((d, fq), jnp.float32),
            pltpu.VMEM((half, fq), jnp.float32),
            pltpu.SemaphoreType.DMA((S,)),
            pltpu.SemaphoreType.DMA((S,)),
            pltpu.SemaphoreType.DMA((S,)),
            pltpu.SemaphoreType.DMA((S,)),
            pltpu.SemaphoreType.DMA((S,)),
            pltpu.SemaphoreType.DMA((S,)),
            pltpu.SemaphoreType.DMA((S,)),
            pltpu.SemaphoreType.DMA((S,)),
        ],
        compiler_params=pltpu.CompilerParams(
            collective_id=0, vmem_limit_bytes=100 * 1024 * 1024
        ),
    )(x, dy)
